# baseline (speedup 1.0000x reference)
import functools

import jax
import jax.numpy as jnp
from jax import lax
from jax.experimental import pallas as pl
from jax.experimental.pallas import tpu as pltpu
from jax.experimental.pallas import tpu_sc as plsc

_L = 16
_NIN = 4
_CPG = 4


def _permute_cols_sc(x_flat, perm_i32, n_rows, n_cols):
    info = plsc.get_sparse_core_info()
    num_cores, num_subcores = info.num_cores, info.num_subcores
    n_workers = num_cores * num_subcores
    rows_per_w = n_rows // n_workers
    chunk_rows = 4
    n_chunks = rows_per_w // chunk_rows
    chunk_elems = chunk_rows * n_cols
    group_elems = _CPG * chunk_elems
    n_groups = n_chunks // _CPG

    mesh = plsc.VectorSubcoreMesh(core_axis_name="c", subcore_axis_name="s")

    @functools.partial(
        pl.kernel,
        out_type=jax.ShapeDtypeStruct((n_rows * n_cols,), jnp.float32),
        mesh=mesh,
        scratch_types=[
            pltpu.VMEM((n_cols,), jnp.int32),
        ]
        + [pltpu.VMEM((chunk_elems,), jnp.float32) for _ in range(_NIN)]
        + [pltpu.VMEM((group_elems,), jnp.float32) for _ in range(2)]
        + [pltpu.SemaphoreType.DMA for _ in range(_NIN + 2)],
        compiler_params=pltpu.CompilerParams(needs_layout_passes=False),
    )
    def k(x_hbm, perm_hbm, out_hbm, perm_v, *bufs_and_sems):
        in_bufs = bufs_and_sems[0:_NIN]
        out_bufs = bufs_and_sems[_NIN:_NIN + 2]
        in_sems = bufs_and_sems[_NIN + 2:2 * _NIN + 2]
        out_sems = bufs_and_sems[2 * _NIN + 2:2 * _NIN + 4]
        wid = lax.axis_index("s") * num_cores + lax.axis_index("c")
        base = wid * rows_per_w * n_cols

        def start_in(g, b):
            pltpu.async_copy(
                x_hbm.at[pl.ds(base + g * chunk_elems, chunk_elems)], in_bufs[b],
                in_sems[b],
            )

        def wait_in(b):
            pltpu.make_async_copy(
                x_hbm.at[pl.ds(0, chunk_elems)], in_bufs[b], in_sems[b]
            ).wait()

        def start_out(grp, bo):
            pltpu.async_copy(
                out_bufs[bo], out_hbm.at[pl.ds(base + grp * group_elems, group_elems)],
                out_sems[bo],
            )

        def wait_out(bo):
            pltpu.make_async_copy(
                out_bufs[bo], out_hbm.at[pl.ds(0, group_elems)], out_sems[bo]
            ).wait()

        def compute(b, bo, off):
            @plsc.parallel_loop(0, n_cols, step=_L, unroll=8)
            def col_body(cbase):
                col = perm_v[pl.ds(cbase, _L)]
                for r in range(chunk_rows):
                    val = plsc.load_gather(in_bufs[b], [col + r * n_cols])
                    out_bufs[bo][pl.ds(off + r * n_cols + cbase, _L)] = val

        def do_group(grp, bo, guard_out):
            if guard_out:
                wait_out(bo)
            for b in range(_CPG):
                g = grp * _CPG + b
                wait_in(b)
                compute(b, bo, b * chunk_elems)

                @pl.when(g + _NIN < n_chunks)
                def _():
                    start_in(g + _NIN, b)

            start_out(grp, bo)

        for b in range(_NIN):
            start_in(b, b)
        pltpu.sync_copy(perm_hbm, perm_v)
        do_group(0, 0, False)
        do_group(1, 1, False)

        def group_body(i, carry):
            grp0 = 2 + 2 * i
            do_group(grp0, 0, True)
            do_group(grp0 + 1, 1, True)
            return carry

        lax.fori_loop(0, (n_groups - 2) // 2, group_body, 0, unroll=1)
        wait_out(0)
        wait_out(1)

    return k(x_flat, perm_i32)


def kernel(x, perm):
    n_rows, n_cols = x.shape
    out_flat = _permute_cols_sc(
        x.reshape(n_rows * n_cols), perm.astype(jnp.int32), n_rows, n_cols
    )
    return out_flat.reshape(n_rows, n_cols)


# speedup vs baseline: 1.0051x; 1.0051x over previous
import functools

import jax
import jax.numpy as jnp
from jax import lax
from jax.experimental import pallas as pl
from jax.experimental.pallas import tpu as pltpu
from jax.experimental.pallas import tpu_sc as plsc

_L = 16
_NBUF = 4


def _permute_cols_sc(x_flat, perm_i32, n_rows, n_cols):
    info = plsc.get_sparse_core_info()
    num_cores, num_subcores = info.num_cores, info.num_subcores
    n_workers = num_cores * num_subcores
    rows_per_w = n_rows // n_workers
    chunk_rows = 4
    n_chunks = rows_per_w // chunk_rows
    chunk_elems = chunk_rows * n_cols

    mesh = plsc.VectorSubcoreMesh(core_axis_name="c", subcore_axis_name="s")

    @functools.partial(
        pl.kernel,
        out_type=jax.ShapeDtypeStruct((n_rows * n_cols,), jnp.float32),
        mesh=mesh,
        scratch_types=[
            pltpu.VMEM((n_cols,), jnp.int32),
        ]
        + [pltpu.VMEM((chunk_elems,), jnp.float32) for _ in range(2 * _NBUF)]
        + [pltpu.SemaphoreType.DMA for _ in range(2 * _NBUF)],
        compiler_params=pltpu.CompilerParams(needs_layout_passes=False),
    )
    def k(x_hbm, perm_hbm, out_hbm, perm_v, *bufs_and_sems):
        in_bufs = bufs_and_sems[0:_NBUF]
        out_bufs = bufs_and_sems[_NBUF:2 * _NBUF]
        in_sems = bufs_and_sems[2 * _NBUF:3 * _NBUF]
        out_sems = bufs_and_sems[3 * _NBUF:4 * _NBUF]
        wid = lax.axis_index("s") * num_cores + lax.axis_index("c")
        base = wid * rows_per_w * n_cols

        def start_in(g, b):
            pltpu.async_copy(
                x_hbm.at[pl.ds(base + g * chunk_elems, chunk_elems)], in_bufs[b],
                in_sems[b],
            )

        def wait_in(b):
            pltpu.make_async_copy(
                x_hbm.at[pl.ds(0, chunk_elems)], in_bufs[b], in_sems[b]
            ).wait()

        def start_out(g, b):
            pltpu.async_copy(
                out_bufs[b], out_hbm.at[pl.ds(base + g * chunk_elems, chunk_elems)],
                out_sems[b],
            )

        def wait_out(b):
            pltpu.make_async_copy(
                out_bufs[b], out_hbm.at[pl.ds(0, chunk_elems)], out_sems[b]
            ).wait()

        def compute(b):
            @plsc.parallel_loop(0, n_cols, step=_L, unroll=8)
            def col_body(cbase):
                col = perm_v[pl.ds(cbase, _L)]
                for r in range(chunk_rows):
                    val = plsc.load_gather(in_bufs[b], [col + r * n_cols])
                    out_bufs[b][pl.ds(r * n_cols + cbase, _L)] = val

        for b in range(_NBUF):
            start_in(b, b)
        pltpu.sync_copy(perm_hbm, perm_v)
        for g in range(_NBUF):
            wait_in(g)
            compute(g)
            start_out(g, g)
            start_in(g + _NBUF, g)

        def chunk_body(i, carry):
            g0 = _NBUF + _NBUF * i
            for b in range(_NBUF):
                g = g0 + b
                wait_in(b)
                wait_out(b)
                compute(b)
                start_out(g, b)

                @pl.when(g + _NBUF < n_chunks)
                def _():
                    start_in(g + _NBUF, b)

            return carry

        lax.fori_loop(0, (n_chunks - _NBUF) // _NBUF, chunk_body, 0, unroll=1)
        for b in range(_NBUF):
            wait_out(b)

    return k(x_flat, perm_i32)


def kernel(x, perm):
    n_rows, n_cols = x.shape
    out_flat = _permute_cols_sc(
        x.reshape(n_rows * n_cols), perm.astype(jnp.int32), n_rows, n_cols
    )
    return out_flat.reshape(n_rows, n_cols)
